# Initial kernel scaffold; baseline (speedup 1.0000x reference)
import jax
import jax.numpy as jnp
from jax.experimental import pallas as pl


def kernel(x, lengths, W_ih, W_hh, b_ih, b_hh):
    raise NotImplementedError("write your pallas kernel here")


# speedup vs baseline: 9.5720x; 9.5720x over previous
import functools

import jax
import jax.numpy as jnp
from jax.experimental import pallas as pl
from jax.experimental.pallas import tpu as pltpu


def _gru_block_kernel(xt_ref, wih_ref, whh_ref, bxi_ref, bhn_ref, len_ref,
                      out_ref, h_ref, xi_ref, *, tblk, b, h, t_total):
    i = pl.program_id(0)

    @pl.when(i == 0)
    def _init():
        h_ref[...] = jnp.zeros_like(h_ref)

    xblk = xt_ref[...].reshape(tblk * b, xt_ref.shape[2])
    xi = jnp.dot(xblk, wih_ref[...], preferred_element_type=jnp.float32)
    xi_ref[...] = (xi + bxi_ref[...]).reshape(tblk, b, 3 * h)

    whh = whh_ref[...]
    bhn = bhn_ref[...]
    lens = len_ref[...]
    t0 = i * tblk

    def step(t, hcur):
        xi_t = xi_ref[t]
        gh = jnp.dot(hcur, whh, preferred_element_type=jnp.float32)
        i_r = xi_t[:, :h]
        i_z = xi_t[:, h:2 * h]
        i_n = xi_t[:, 2 * h:]
        h_r = gh[:, :h]
        h_z = gh[:, h:2 * h]
        h_n = gh[:, 2 * h:] + bhn
        r = jax.nn.sigmoid(i_r + h_r)
        z = jax.nn.sigmoid(i_z + h_z)
        n = jnp.tanh(i_n + r * h_n)
        hnew = (1.0 - z) * n + z * hcur
        mask = (lens > (t0 + t)).astype(hnew.dtype)
        out_ref[t] = hnew * mask
        return hnew

    h_ref[...] = jax.lax.fori_loop(0, tblk, step, h_ref[...])


def kernel(x, lengths, W_ih, W_hh, b_ih, b_hh):
    B, C, T = x.shape
    H = W_hh.shape[1]
    TBLK = 256
    assert T % TBLK == 0

    xt = jnp.transpose(x, (2, 0, 1))
    wih_t = W_ih.T
    whh_t = W_hh.T
    bxi = (b_ih + jnp.concatenate([b_hh[:2 * H], jnp.zeros((H,), b_hh.dtype)])
           ).reshape(1, 3 * H)
    bhn = b_hh[2 * H:].reshape(1, H)
    lens2d = jnp.broadcast_to(lengths.astype(jnp.int32)[:, None], (B, H))

    grid = (T // TBLK,)
    out_tbh = pl.pallas_call(
        functools.partial(_gru_block_kernel, tblk=TBLK, b=B, h=H, t_total=T),
        grid=grid,
        in_specs=[
            pl.BlockSpec((TBLK, B, C), lambda i: (i, 0, 0)),
            pl.BlockSpec((C, 3 * H), lambda i: (0, 0)),
            pl.BlockSpec((H, 3 * H), lambda i: (0, 0)),
            pl.BlockSpec((1, 3 * H), lambda i: (0, 0)),
            pl.BlockSpec((1, H), lambda i: (0, 0)),
            pl.BlockSpec((B, H), lambda i: (0, 0)),
        ],
        out_specs=pl.BlockSpec((TBLK, B, H), lambda i: (i, 0, 0)),
        out_shape=jax.ShapeDtypeStruct((T, B, H), x.dtype),
        scratch_shapes=[
            pltpu.VMEM((B, H), jnp.float32),
            pltpu.VMEM((TBLK, B, 3 * H), jnp.float32),
        ],
        compiler_params=pltpu.CompilerParams(
            dimension_semantics=("arbitrary",),
        ),
    )(xt, wih_t, whh_t, bxi, bhn, lens2d)

    out = jnp.transpose(out_tbh, (1, 2, 0))
    return (out, lengths)


# speedup vs baseline: 9.5753x; 1.0003x over previous
import functools

import jax
import jax.numpy as jnp
from jax.experimental import pallas as pl
from jax.experimental.pallas import tpu as pltpu


def _gru_block_kernel(xt_ref, wih_ref, whh_ref, bxi_ref, bhn_ref, len_ref,
                      out_ref, h_ref, xi_ref, *, tblk, b, h, t_total):
    i = pl.program_id(0)

    @pl.when(i == 0)
    def _init():
        h_ref[...] = jnp.zeros_like(h_ref)

    xblk = xt_ref[...].reshape(tblk * b, xt_ref.shape[2])
    xi = jnp.dot(xblk.astype(jnp.bfloat16), wih_ref[...],
                 preferred_element_type=jnp.float32)
    xi_ref[...] = (xi + bxi_ref[...]).reshape(tblk, b, 3 * h)

    whh = whh_ref[...]
    bhn = bhn_ref[...]
    lens = len_ref[...]
    t0 = i * tblk

    def step(t, hcur):
        xi_t = xi_ref[t]
        gh = jnp.dot(hcur.astype(jnp.bfloat16), whh,
                     preferred_element_type=jnp.float32)
        i_r = xi_t[:, :h]
        i_z = xi_t[:, h:2 * h]
        i_n = xi_t[:, 2 * h:]
        h_r = gh[:, :h]
        h_z = gh[:, h:2 * h]
        h_n = gh[:, 2 * h:] + bhn
        r = jax.nn.sigmoid(i_r + h_r)
        z = jax.nn.sigmoid(i_z + h_z)
        n = jnp.tanh(i_n + r * h_n)
        hnew = (1.0 - z) * n + z * hcur
        mask = (lens > (t0 + t)).astype(hnew.dtype)
        out_ref[t] = hnew * mask
        return hnew

    h_ref[...] = jax.lax.fori_loop(0, tblk, step, h_ref[...])


def kernel(x, lengths, W_ih, W_hh, b_ih, b_hh):
    B, C, T = x.shape
    H = W_hh.shape[1]
    TBLK = 256
    assert T % TBLK == 0

    xt = jnp.transpose(x, (2, 0, 1))
    wih_t = W_ih.T.astype(jnp.bfloat16)
    whh_t = W_hh.T.astype(jnp.bfloat16)
    bxi = (b_ih + jnp.concatenate([b_hh[:2 * H], jnp.zeros((H,), b_hh.dtype)])
           ).reshape(1, 3 * H)
    bhn = b_hh[2 * H:].reshape(1, H)
    lens2d = jnp.broadcast_to(lengths.astype(jnp.int32)[:, None], (B, H))

    grid = (T // TBLK,)
    out_tbh = pl.pallas_call(
        functools.partial(_gru_block_kernel, tblk=TBLK, b=B, h=H, t_total=T),
        grid=grid,
        in_specs=[
            pl.BlockSpec((TBLK, B, C), lambda i: (i, 0, 0)),
            pl.BlockSpec((C, 3 * H), lambda i: (0, 0)),
            pl.BlockSpec((H, 3 * H), lambda i: (0, 0)),
            pl.BlockSpec((1, 3 * H), lambda i: (0, 0)),
            pl.BlockSpec((1, H), lambda i: (0, 0)),
            pl.BlockSpec((B, H), lambda i: (0, 0)),
        ],
        out_specs=pl.BlockSpec((TBLK, B, H), lambda i: (i, 0, 0)),
        out_shape=jax.ShapeDtypeStruct((T, B, H), x.dtype),
        scratch_shapes=[
            pltpu.VMEM((B, H), jnp.float32),
            pltpu.VMEM((TBLK, B, 3 * H), jnp.float32),
        ],
        compiler_params=pltpu.CompilerParams(
            dimension_semantics=("arbitrary",),
        ),
    )(xt, wih_t, whh_t, bxi, bhn, lens2d)

    out = jnp.transpose(out_tbh, (1, 2, 0))
    return (out, lengths)


# speedup vs baseline: 10.7115x; 1.1187x over previous
import functools

import jax
import jax.numpy as jnp
from jax.experimental import pallas as pl
from jax.experimental.pallas import tpu as pltpu


def _gru_block_kernel(len_sref, x_ref, wih_ref, whh_ref, bxi_ref, bhn_ref,
                      len_ref, out_ref, h_ref, xi_ref, *, tblk, b, h):
    i = pl.program_id(0)

    @pl.when(i == 0)
    def _init():
        h_ref[...] = jnp.zeros_like(h_ref)

    t0 = i * tblk
    maxlen = len_sref[0]
    nrem = jnp.clip(maxlen - t0, 0, tblk)
    nsteps = jnp.minimum((nrem + 1) & ~1, tblk)

    out_ref[...] = jnp.zeros_like(out_ref)

    @pl.when(nrem > 0)
    def _work():
        xt = jnp.transpose(x_ref[...], (2, 0, 1))
        xblk = xt.reshape(tblk * b, x_ref.shape[1])
        xi = jnp.dot(xblk.astype(jnp.bfloat16), wih_ref[...],
                     preferred_element_type=jnp.float32)
        xi_ref[...] = (xi + bxi_ref[...]).reshape(tblk, b, 3 * h)

        whh = whh_ref[...]
        bhn = bhn_ref[...]
        lens = len_ref[...]

        def step(t, hcur):
            xi_t = xi_ref[t]
            gh = jnp.dot(hcur.astype(jnp.bfloat16), whh,
                         preferred_element_type=jnp.float32)
            tr = jnp.tanh(xi_t[:, :h] + gh[:, :h])
            tz = jnp.tanh(xi_t[:, h:2 * h] + gh[:, h:2 * h])
            hn2 = 0.5 * gh[:, 2 * h:] + bhn
            n = jnp.tanh((xi_t[:, 2 * h:] + hn2) + tr * hn2)
            zz = 0.5 + 0.5 * tz
            hnew = n + zz * (hcur - n)
            mask = (lens > (t0 + t)).astype(hnew.dtype)
            out_ref[t] = hnew * mask
            return hnew

        def step2(j, hcur):
            return step(2 * j + 1, step(2 * j, hcur))

        h_ref[...] = jax.lax.fori_loop(0, nsteps // 2, step2, h_ref[...])


def kernel(x, lengths, W_ih, W_hh, b_ih, b_hh):
    B, C, T = x.shape
    H = W_hh.shape[1]
    TBLK = 256
    assert T % TBLK == 0

    scale = jnp.concatenate([jnp.full((2 * H,), 0.5, jnp.float32),
                             jnp.ones((H,), jnp.float32)])
    wih_t = (W_ih.T * scale[None, :]).astype(jnp.bfloat16)
    whh_t = (W_hh.T * scale[None, :]).astype(jnp.bfloat16)
    bxi = ((b_ih + jnp.concatenate([b_hh[:2 * H],
                                    jnp.zeros((H,), b_hh.dtype)])) * scale
           ).reshape(1, 3 * H)
    bhn = (0.5 * b_hh[2 * H:]).reshape(1, H)
    lens_i32 = lengths.astype(jnp.int32)
    lens2d = jnp.broadcast_to(lens_i32[:, None], (B, H))

    grid_spec = pltpu.PrefetchScalarGridSpec(
        num_scalar_prefetch=1,
        grid=(T // TBLK,),
        in_specs=[
            pl.BlockSpec((B, C, TBLK), lambda i, sref: (0, 0, i)),
            pl.BlockSpec((C, 3 * H), lambda i, sref: (0, 0)),
            pl.BlockSpec((H, 3 * H), lambda i, sref: (0, 0)),
            pl.BlockSpec((1, 3 * H), lambda i, sref: (0, 0)),
            pl.BlockSpec((1, H), lambda i, sref: (0, 0)),
            pl.BlockSpec((B, H), lambda i, sref: (0, 0)),
        ],
        out_specs=pl.BlockSpec((TBLK, B, H), lambda i, sref: (i, 0, 0)),
        scratch_shapes=[
            pltpu.VMEM((B, H), jnp.float32),
            pltpu.VMEM((TBLK, B, 3 * H), jnp.float32),
        ],
    )
    out_tbh = pl.pallas_call(
        functools.partial(_gru_block_kernel, tblk=TBLK, b=B, h=H),
        grid_spec=grid_spec,
        out_shape=jax.ShapeDtypeStruct((T, B, H), x.dtype),
        compiler_params=pltpu.CompilerParams(
            dimension_semantics=("arbitrary",),
        ),
    )(lens_i32, x, wih_t, whh_t, bxi, bhn, lens2d)

    out = jnp.transpose(out_tbh, (1, 2, 0))
    return (out, lengths)


# speedup vs baseline: 11.1634x; 1.0422x over previous
import functools

import jax
import jax.numpy as jnp
from jax.experimental import pallas as pl
from jax.experimental.pallas import tpu as pltpu


def _gru_block_kernel(len_sref, x_ref, wih_ref, whh_ref, bxi_ref, bhn_ref,
                      len_ref, out_ref, h_ref, xi_ref, *, tblk, b, h):
    i = pl.program_id(0)

    @pl.when(i == 0)
    def _init():
        h_ref[...] = jnp.zeros_like(h_ref)

    t0 = i * tblk
    maxlen = len_sref[0]
    nrem = jnp.clip(maxlen - t0, 0, tblk)
    nsteps = jnp.minimum((nrem + 3) & ~3, tblk)

    @pl.when(nrem < tblk)
    def _zero():
        out_ref[...] = jnp.zeros_like(out_ref)

    @pl.when(nrem > 0)
    def _work():
        xt = jnp.transpose(x_ref[...], (2, 0, 1))
        xblk = xt.reshape(tblk * b, x_ref.shape[1])
        xi = jnp.dot(xblk.astype(jnp.bfloat16), wih_ref[...],
                     preferred_element_type=jnp.float32)
        xi_ref[...] = (xi + bxi_ref[...]).reshape(tblk, b, 3 * h)

        whh = whh_ref[...]
        bhn = bhn_ref[...]
        lens = len_ref[...]

        def step(t, hcur):
            xi_t = xi_ref[t]
            gh = jnp.dot(hcur.astype(jnp.bfloat16), whh,
                         preferred_element_type=jnp.float32)
            tr = jnp.tanh(xi_t[:, :h] + gh[:, :h])
            tz = jnp.tanh(xi_t[:, h:2 * h] + gh[:, h:2 * h])
            hn2 = 0.5 * gh[:, 2 * h:] + bhn
            n = jnp.tanh((xi_t[:, 2 * h:] + hn2) + tr * hn2)
            zn = 0.5 - 0.5 * tz
            zh = (0.5 + 0.5 * tz) * hcur
            hnew = n * zn + zh
            mask = (lens > (t0 + t)).astype(hnew.dtype)
            out_ref[t] = hnew * mask
            return hnew

        def step4(j, hcur):
            return step(4 * j + 3, step(4 * j + 2,
                        step(4 * j + 1, step(4 * j, hcur))))

        h_ref[...] = jax.lax.fori_loop(0, nsteps // 4, step4, h_ref[...])


def kernel(x, lengths, W_ih, W_hh, b_ih, b_hh):
    B, C, T = x.shape
    H = W_hh.shape[1]
    TBLK = 512
    assert T % TBLK == 0

    scale = jnp.concatenate([jnp.full((2 * H,), 0.5, jnp.float32),
                             jnp.ones((H,), jnp.float32)])
    wih_t = (W_ih.T * scale[None, :]).astype(jnp.bfloat16)
    whh_t = (W_hh.T * scale[None, :]).astype(jnp.bfloat16)
    bxi = ((b_ih + jnp.concatenate([b_hh[:2 * H],
                                    jnp.zeros((H,), b_hh.dtype)])) * scale
           ).reshape(1, 3 * H)
    bhn = (0.5 * b_hh[2 * H:]).reshape(1, H)
    lens_i32 = lengths.astype(jnp.int32)
    lens2d = jnp.broadcast_to(lens_i32[:, None], (B, H))

    grid_spec = pltpu.PrefetchScalarGridSpec(
        num_scalar_prefetch=1,
        grid=(T // TBLK,),
        in_specs=[
            pl.BlockSpec((B, C, TBLK), lambda i, sref: (0, 0, i)),
            pl.BlockSpec((C, 3 * H), lambda i, sref: (0, 0)),
            pl.BlockSpec((H, 3 * H), lambda i, sref: (0, 0)),
            pl.BlockSpec((1, 3 * H), lambda i, sref: (0, 0)),
            pl.BlockSpec((1, H), lambda i, sref: (0, 0)),
            pl.BlockSpec((B, H), lambda i, sref: (0, 0)),
        ],
        out_specs=pl.BlockSpec((TBLK, B, H), lambda i, sref: (i, 0, 0)),
        scratch_shapes=[
            pltpu.VMEM((B, H), jnp.float32),
            pltpu.VMEM((TBLK, B, 3 * H), jnp.float32),
        ],
    )
    out_tbh = pl.pallas_call(
        functools.partial(_gru_block_kernel, tblk=TBLK, b=B, h=H),
        grid_spec=grid_spec,
        out_shape=jax.ShapeDtypeStruct((T, B, H), x.dtype),
        compiler_params=pltpu.CompilerParams(
            dimension_semantics=("arbitrary",),
        ),
    )(lens_i32, x, wih_t, whh_t, bxi, bhn, lens2d)

    out = jnp.transpose(out_tbh, (1, 2, 0))
    return (out, lengths)
